# Initial kernel scaffold; baseline (speedup 1.0000x reference)
import jax
import jax.numpy as jnp
from jax.experimental import pallas as pl


def kernel(input):
    raise NotImplementedError("write your pallas kernel here")


# speedup vs baseline: 15.0347x; 15.0347x over previous
import functools

import jax
import jax.numpy as jnp
from jax import lax
from jax.experimental import pallas as pl
from jax.experimental.pallas import tpu as pltpu
from jax.experimental.pallas import tpu_sc as plsc

N, C, H, W = 16, 768, 32, 32
M = 4
HW = H * W
G = (N * C) // M
NC, NS, L = 2, 16, 16
NW = NC * NS
GPW = G // NW
SLAB = 8
NSLAB = GPW // SLAB


def _mask_step(ibuf, obuf, g, off):
    x0 = ibuf[g, 0, pl.ds(off, L)]
    x1 = ibuf[g, 1, pl.ds(off, L)]
    x2 = ibuf[g, 2, pl.ds(off, L)]
    x3 = ibuf[g, 3, pl.ds(off, L)]
    a0, a1, a2, a3 = jnp.abs(x0), jnp.abs(x1), jnp.abs(x2), jnp.abs(x3)
    one = jnp.ones((L,), jnp.float32)
    zero = jnp.zeros((L,), jnp.float32)
    f01 = jnp.where(a0 <= a1, one, zero)
    f02 = jnp.where(a0 <= a2, one, zero)
    f03 = jnp.where(a0 <= a3, one, zero)
    f12 = jnp.where(a1 <= a2, one, zero)
    f13 = jnp.where(a1 <= a3, one, zero)
    f23 = jnp.where(a2 <= a3, one, zero)
    k0 = (f01 + f02) + f03 <= 1.0
    k1 = f01 - (f12 + f13) >= 0.0
    k2 = (f02 + f12) - f23 >= 1.0
    k3 = (f03 + f13) + f23 >= 2.0
    obuf[g, 0, pl.ds(off, L)] = jnp.where(k0, x0, zero)
    obuf[g, 1, pl.ds(off, L)] = jnp.where(k1, x1, zero)
    obuf[g, 2, pl.ds(off, L)] = jnp.where(k2, x2, zero)
    obuf[g, 3, pl.ds(off, L)] = jnp.where(k3, x3, zero)


def _compute_slab(ibuf, obuf):
    def gloop(g, carry):
        def iloop(i, carry2):
            for u in range(4):
                _mask_step(ibuf, obuf, g, (i * 4 + u) * L)
            return carry2
        lax.fori_loop(0, HW // (4 * L), iloop, carry)
        return carry
    lax.fori_loop(0, SLAB, gloop, 0)


@functools.partial(
    pl.kernel,
    mesh=plsc.VectorSubcoreMesh(core_axis_name="c", subcore_axis_name="s"),
    out_type=jax.ShapeDtypeStruct((G, M, HW), jnp.float32),
    scratch_types=[
        pltpu.VMEM((SLAB, M, HW), jnp.float32),
        pltpu.VMEM((SLAB, M, HW), jnp.float32),
    ],
)
def _nm_sparsity_sc(x_hbm, o_hbm, ibuf, obuf):
    wid = lax.axis_index("s") * NC + lax.axis_index("c")
    for s in range(NSLAB):
        gbase = wid * GPW + s * SLAB
        pltpu.sync_copy(x_hbm.at[pl.ds(gbase, SLAB)], ibuf)
        _compute_slab(ibuf, obuf)
        pltpu.sync_copy(obuf, o_hbm.at[pl.ds(gbase, SLAB)])


def kernel(input):
    x = input.reshape(G, M, HW)
    out = _nm_sparsity_sc(x)
    return out.reshape(N, C, H, W)


# speedup vs baseline: 27.6585x; 1.8396x over previous
import functools

import jax
import jax.numpy as jnp
from jax import lax
from jax.experimental import pallas as pl
from jax.experimental.pallas import tpu as pltpu
from jax.experimental.pallas import tpu_sc as plsc

N, C, H, W = 16, 768, 32, 32
R = N * H * W
NC, NS, L = 2, 16, 16
NW = NC * NS
RPW = R // NW
RS = 32
NSLAB = RPW // RS
UNROLL = 4


def _compute_slab(ibuf, obuf):
    iota = lax.iota(jnp.int32, L)
    pos = iota & 3
    base4 = iota & (-4)
    onei = jnp.ones((L,), jnp.int32)
    zeroi = jnp.zeros((L,), jnp.int32)
    zerof = jnp.zeros((L,), jnp.float32)
    perms = []
    ties = []
    for k in (1, 2, 3):
        npos = (iota + k) & 3
        perms.append(base4 | npos)
        ties.append(jnp.where(npos < pos, onei, zeroi))
    p1, p2, p3 = perms
    t1, t2, t3 = ties
    msk = jnp.int32(0x7FFFFFFF)

    def rbody(r, carry):
        def cbody(j, carry2):
            for u in range(UNROLL):
                c = (j * UNROLL + u) * L
                v = ibuf[r, pl.ds(c, L)]
                ia = lax.bitcast_convert_type(v, jnp.int32) & msk
                b1 = ia.at[p1].get(mode="promise_in_bounds")
                b2 = ia.at[p2].get(mode="promise_in_bounds")
                b3 = ia.at[p3].get(mode="promise_in_bounds")
                s1 = b1 - (ia + t1)
                s2 = b2 - (ia + t2)
                s3 = b3 - (ia + t3)
                m = (s1 & s2) | (s3 & (s1 | s2))
                obuf[r, pl.ds(c, L)] = jnp.where(m < 0, v, zerof)
            return carry2

        lax.fori_loop(0, C // (UNROLL * L), cbody, carry)
        return carry

    lax.fori_loop(0, RS, rbody, 0)


@functools.partial(
    pl.kernel,
    mesh=plsc.VectorSubcoreMesh(core_axis_name="c", subcore_axis_name="s"),
    out_type=jax.ShapeDtypeStruct((R, C), jnp.float32),
    scratch_types=[
        pltpu.VMEM((RS, C), jnp.float32),
        pltpu.VMEM((RS, C), jnp.float32),
    ],
    compiler_params=pltpu.CompilerParams(use_tc_tiling_on_sc=True),
)
def _nm_sparsity_sc(x_hbm, o_hbm, ibuf, obuf):
    wid = lax.axis_index("s") * NC + lax.axis_index("c")
    for s in range(NSLAB):
        r0 = wid * RPW + s * RS
        pltpu.sync_copy(x_hbm.at[pl.ds(r0, RS)], ibuf)
        _compute_slab(ibuf, obuf)
        pltpu.sync_copy(obuf, o_hbm.at[pl.ds(r0, RS)])


def kernel(input):
    x = input.transpose(0, 2, 3, 1).reshape(R, C)
    out = _nm_sparsity_sc(x)
    return out.reshape(N, H, W, C).transpose(0, 3, 1, 2)


# speedup vs baseline: 28.8829x; 1.0443x over previous
import functools

import jax
import jax.numpy as jnp
from jax import lax
from jax.experimental import pallas as pl
from jax.experimental.pallas import tpu as pltpu
from jax.experimental.pallas import tpu_sc as plsc

N, C, H, W = 16, 768, 32, 32
R = N * H * W
NC, NS, L = 2, 16, 16
NW = NC * NS
RPW = R // NW
RS = 32
NSLAB = RPW // RS
UNROLL = 4


def _compute_slab(ibuf, obuf):
    iota = lax.iota(jnp.int32, L)
    pos = iota & 3
    base4 = iota & (-4)
    onei = jnp.ones((L,), jnp.int32)
    zeroi = jnp.zeros((L,), jnp.int32)
    zerof = jnp.zeros((L,), jnp.float32)
    p1 = base4 | ((iota + 1) & 3)
    p2 = base4 | ((iota + 2) & 3)
    p3 = base4 | ((iota + 3) & 3)
    t1 = jnp.where(((iota + 1) & 3) < pos, onei, zeroi)
    t2 = jnp.where(((iota + 2) & 3) < pos, onei, zeroi)
    msk = jnp.int32(0x7FFFFFFF)

    def rbody(r, carry):
        def cbody(j, carry2):
            for u in range(UNROLL):
                c = (j * UNROLL + u) * L
                v = ibuf[r, pl.ds(c, L)]
                ia = lax.bitcast_convert_type(v, jnp.int32) & msk
                b1 = ia.at[p1].get(mode="promise_in_bounds")
                b2 = ia.at[p2].get(mode="promise_in_bounds")
                below1 = lax.shift_right_logical(b1 - (ia + t1), 31)
                below2 = lax.shift_right_logical(b2 - (ia + t2), 31)
                b3p = below1.at[p3].get(mode="promise_in_bounds")
                rank = (below1 + below2) - b3p
                obuf[r, pl.ds(c, L)] = jnp.where(rank >= 1, v, zerof)
            return carry2

        lax.fori_loop(0, C // (UNROLL * L), cbody, carry)
        return carry

    lax.fori_loop(0, RS, rbody, 0)


@functools.partial(
    pl.kernel,
    mesh=plsc.VectorSubcoreMesh(core_axis_name="c", subcore_axis_name="s"),
    out_type=jax.ShapeDtypeStruct((R, C), jnp.float32),
    scratch_types=[
        pltpu.VMEM((RS, C), jnp.float32),
        pltpu.VMEM((RS, C), jnp.float32),
        pltpu.VMEM((RS, C), jnp.float32),
        pltpu.VMEM((RS, C), jnp.float32),
        pltpu.SemaphoreType.DMA,
        pltpu.SemaphoreType.DMA,
        pltpu.SemaphoreType.DMA,
        pltpu.SemaphoreType.DMA,
    ],
    compiler_params=pltpu.CompilerParams(use_tc_tiling_on_sc=True),
)
def _nm_sparsity_sc(x_hbm, o_hbm, ib0, ib1, ob0, ob1, is0, is1, os0, os1):
    wid = lax.axis_index("s") * NC + lax.axis_index("c")
    base = wid * RPW
    ibufs, obufs = (ib0, ib1), (ob0, ob1)
    isems, osems = (is0, is1), (os0, os1)

    def start_in(s):
        b = s & 1
        return pltpu.async_copy(
            x_hbm.at[pl.ds(base + s * RS, RS)], ibufs[b], isems[b])

    def start_out(s):
        b = s & 1
        return pltpu.async_copy(
            obufs[b], o_hbm.at[pl.ds(base + s * RS, RS)], osems[b])

    h_in = {0: start_in(0)}
    h_out = {}
    for s in range(NSLAB):
        b = s & 1
        if s + 1 < NSLAB:
            h_in[s + 1] = start_in(s + 1)
        h_in[s].wait()
        if s >= 2:
            h_out[s - 2].wait()
        _compute_slab(ibufs[b], obufs[b])
        h_out[s] = start_out(s)
    h_out[NSLAB - 2].wait()
    h_out[NSLAB - 1].wait()


def kernel(input):
    x = input.transpose(0, 2, 3, 1).reshape(R, C)
    out = _nm_sparsity_sc(x)
    return out.reshape(N, H, W, C).transpose(0, 3, 1, 2)


# speedup vs baseline: 71.2309x; 2.4662x over previous
import functools

import jax
import jax.numpy as jnp
from jax import lax
from jax.experimental import pallas as pl
from jax.experimental.pallas import tpu as pltpu
from jax.experimental.pallas import tpu_sc as plsc

N, C, H, W = 16, 768, 32, 32
R = N * H * W
NC, NS, L = 2, 16, 16
NW = NC * NS
RPW = R // NW
RS = 32
NSLAB = RPW // RS
UNROLL = 8


def _compute_slab(ibuf, obuf):
    iota = lax.iota(jnp.int32, L)
    pos = iota & 3
    base4 = iota & (-4)
    onei = jnp.ones((L,), jnp.int32)
    zeroi = jnp.zeros((L,), jnp.int32)
    zerof = jnp.zeros((L,), jnp.float32)
    p1 = base4 | ((iota + 1) & 3)
    p2 = base4 | ((iota + 2) & 3)
    p3 = base4 | ((iota + 3) & 3)
    t1 = jnp.where(((iota + 1) & 3) < pos, onei, zeroi)
    t2 = jnp.where(((iota + 2) & 3) < pos, onei, zeroi)
    msk = jnp.int32(0x7FFFFFFF)

    def rbody(r, carry):
        def cbody(j, carry2):
            U = range(UNROLL)
            off = [(j * UNROLL + u) * L for u in U]
            v = [ibuf[r, pl.ds(off[u], L)] for u in U]
            ia = [lax.bitcast_convert_type(v[u], jnp.int32) & msk for u in U]
            b1 = [ia[u].at[p1].get(mode="promise_in_bounds") for u in U]
            b2 = [ia[u].at[p2].get(mode="promise_in_bounds") for u in U]
            below1 = [lax.shift_right_logical(b1[u] - (ia[u] + t1), 31)
                      for u in U]
            below2 = [lax.shift_right_logical(b2[u] - (ia[u] + t2), 31)
                      for u in U]
            b3p = [below1[u].at[p3].get(mode="promise_in_bounds") for u in U]
            rank = [(below1[u] + below2[u]) - b3p[u] for u in U]
            for u in U:
                obuf[r, pl.ds(off[u], L)] = jnp.where(rank[u] >= 1, v[u],
                                                      zerof)
            return carry2

        lax.fori_loop(0, C // (UNROLL * L), cbody, carry)
        return carry

    lax.fori_loop(0, RS, rbody, 0)


@functools.partial(
    pl.kernel,
    mesh=plsc.VectorSubcoreMesh(core_axis_name="c", subcore_axis_name="s"),
    out_type=jax.ShapeDtypeStruct((R, C), jnp.float32),
    scratch_types=[
        pltpu.VMEM((RS, C), jnp.float32),
        pltpu.VMEM((RS, C), jnp.float32),
        pltpu.VMEM((RS, C), jnp.float32),
        pltpu.VMEM((RS, C), jnp.float32),
        pltpu.SemaphoreType.DMA,
        pltpu.SemaphoreType.DMA,
        pltpu.SemaphoreType.DMA,
        pltpu.SemaphoreType.DMA,
    ],
    compiler_params=pltpu.CompilerParams(use_tc_tiling_on_sc=True),
)
def _nm_sparsity_sc(x_hbm, o_hbm, ib0, ib1, ob0, ob1, is0, is1, os0, os1):
    wid = lax.axis_index("s") * NC + lax.axis_index("c")
    base = wid * RPW
    ibufs, obufs = (ib0, ib1), (ob0, ob1)
    isems, osems = (is0, is1), (os0, os1)

    def start_in(s):
        b = s & 1
        return pltpu.async_copy(
            x_hbm.at[pl.ds(base + s * RS, RS)], ibufs[b], isems[b])

    def start_out(s):
        b = s & 1
        return pltpu.async_copy(
            obufs[b], o_hbm.at[pl.ds(base + s * RS, RS)], osems[b])

    h_in = {0: start_in(0)}
    h_out = {}
    for s in range(NSLAB):
        b = s & 1
        if s + 1 < NSLAB:
            h_in[s + 1] = start_in(s + 1)
        h_in[s].wait()
        if s >= 2:
            h_out[s - 2].wait()
        _compute_slab(ibufs[b], obufs[b])
        h_out[s] = start_out(s)
    h_out[NSLAB - 2].wait()
    h_out[NSLAB - 1].wait()


def kernel(input):
    x = input.transpose(0, 2, 3, 1).reshape(R, C)
    out = _nm_sparsity_sc(x)
    return out.reshape(N, H, W, C).transpose(0, 3, 1, 2)


# speedup vs baseline: 75.7942x; 1.0641x over previous
import functools

import jax
import jax.numpy as jnp
from jax import lax
from jax.experimental import pallas as pl
from jax.experimental.pallas import tpu as pltpu
from jax.experimental.pallas import tpu_sc as plsc

N, C, H, W = 16, 768, 32, 32
R = N * H * W
NC, NS, L = 2, 16, 16
NW = NC * NS
RPW = R // NW
RS = 32
NSLAB = RPW // RS
UNROLL = 12


def _compute_slab(ibuf, obuf):
    iota = lax.iota(jnp.int32, L)
    pos = iota & 3
    base4 = iota & (-4)
    onei = jnp.ones((L,), jnp.int32)
    zeroi = jnp.zeros((L,), jnp.int32)
    zerof = jnp.zeros((L,), jnp.float32)
    p1 = base4 | ((iota + 1) & 3)
    p2 = base4 | ((iota + 2) & 3)
    p3 = base4 | ((iota + 3) & 3)
    t1 = jnp.where(((iota + 1) & 3) < pos, onei, zeroi)
    t2 = jnp.where(((iota + 2) & 3) < pos, onei, zeroi)
    msk = jnp.int32(0x7FFFFFFF)

    def rbody(r, carry):
        def cbody(j, carry2):
            U = range(UNROLL)
            off = [(j * UNROLL + u) * L for u in U]
            v = [ibuf[r, pl.ds(off[u], L)] for u in U]
            ia = [lax.bitcast_convert_type(v[u], jnp.int32) & msk for u in U]
            b1 = [ia[u].at[p1].get(mode="promise_in_bounds") for u in U]
            b2 = [ia[u].at[p2].get(mode="promise_in_bounds") for u in U]
            below1 = [lax.shift_right_logical(b1[u] - (ia[u] + t1), 31)
                      for u in U]
            below2 = [lax.shift_right_logical(b2[u] - (ia[u] + t2), 31)
                      for u in U]
            b3p = [below1[u].at[p3].get(mode="promise_in_bounds") for u in U]
            rank = [(below1[u] + below2[u]) - b3p[u] for u in U]
            for u in U:
                obuf[r, pl.ds(off[u], L)] = jnp.where(rank[u] >= 1, v[u],
                                                      zerof)
            return carry2

        lax.fori_loop(0, C // (UNROLL * L), cbody, carry)
        return carry

    lax.fori_loop(0, RS, rbody, 0)


@functools.partial(
    pl.kernel,
    mesh=plsc.VectorSubcoreMesh(core_axis_name="c", subcore_axis_name="s"),
    out_type=jax.ShapeDtypeStruct((R, C), jnp.float32),
    scratch_types=[
        pltpu.VMEM((RS, C), jnp.float32),
        pltpu.VMEM((RS, C), jnp.float32),
        pltpu.VMEM((RS, C), jnp.float32),
        pltpu.VMEM((RS, C), jnp.float32),
        pltpu.SemaphoreType.DMA,
        pltpu.SemaphoreType.DMA,
        pltpu.SemaphoreType.DMA,
        pltpu.SemaphoreType.DMA,
    ],
    compiler_params=pltpu.CompilerParams(use_tc_tiling_on_sc=True),
)
def _nm_sparsity_sc(x_hbm, o_hbm, ib0, ib1, ob0, ob1, is0, is1, os0, os1):
    wid = lax.axis_index("s") * NC + lax.axis_index("c")
    base = wid * RPW
    ibufs, obufs = (ib0, ib1), (ob0, ob1)
    isems, osems = (is0, is1), (os0, os1)

    def start_in(s):
        b = s & 1
        return pltpu.async_copy(
            x_hbm.at[pl.ds(base + s * RS, RS)], ibufs[b], isems[b])

    def start_out(s):
        b = s & 1
        return pltpu.async_copy(
            obufs[b], o_hbm.at[pl.ds(base + s * RS, RS)], osems[b])

    h_in = {0: start_in(0)}
    h_out = {}
    for s in range(NSLAB):
        b = s & 1
        if s + 1 < NSLAB:
            h_in[s + 1] = start_in(s + 1)
        h_in[s].wait()
        if s >= 2:
            h_out[s - 2].wait()
        _compute_slab(ibufs[b], obufs[b])
        h_out[s] = start_out(s)
    h_out[NSLAB - 2].wait()
    h_out[NSLAB - 1].wait()


def kernel(input):
    x = input.transpose(0, 2, 3, 1).reshape(R, C)
    out = _nm_sparsity_sc(x)
    return out.reshape(N, H, W, C).transpose(0, 3, 1, 2)


# speedup vs baseline: 78.5448x; 1.0363x over previous
import functools

import jax
import jax.numpy as jnp
from jax import lax
from jax.experimental import pallas as pl
from jax.experimental.pallas import tpu as pltpu
from jax.experimental.pallas import tpu_sc as plsc

N, C, H, W = 16, 768, 32, 32
R = N * H * W
NC, NS, L = 2, 16, 16
NW = NC * NS
RPW = R // NW
RS = 32
NSLAB = RPW // RS
UNROLL = 12


def _compute_slab(ibuf, obuf):
    iota = lax.iota(jnp.int32, L)
    pos = iota & 3
    base4 = iota & (-4)
    onei = jnp.ones((L,), jnp.int32)
    zeroi = jnp.zeros((L,), jnp.int32)
    zerof = jnp.zeros((L,), jnp.float32)
    p1 = base4 | ((iota + 1) & 3)
    p2 = base4 | ((iota + 2) & 3)
    p3 = base4 | ((iota + 3) & 3)
    t1 = jnp.where(((iota + 1) & 3) < pos, onei, zeroi)
    t2 = jnp.where(((iota + 2) & 3) < pos, onei, zeroi)
    msk = jnp.int32(0x7FFFFFFF)

    def rbody(r, carry):
        def cbody(j, carry2):
            U = range(UNROLL)
            off = [(j * UNROLL + u) * L for u in U]
            v = [ibuf[r, pl.ds(off[u], L)] for u in U]
            ia = [lax.bitcast_convert_type(v[u], jnp.int32) & msk for u in U]
            b1 = [ia[u].at[p1].get(mode="promise_in_bounds") for u in U]
            b2 = [ia[u].at[p2].get(mode="promise_in_bounds") for u in U]
            below1 = [lax.shift_right_logical(b1[u] - (ia[u] + t1), 31)
                      for u in U]
            below2 = [lax.shift_right_logical(b2[u] - (ia[u] + t2), 31)
                      for u in U]
            b3p = [below1[u].at[p3].get(mode="promise_in_bounds") for u in U]
            rank = [(below1[u] + below2[u]) - b3p[u] for u in U]
            for u in U:
                obuf[r, pl.ds(off[u], L)] = jnp.where(rank[u] >= 1, v[u],
                                                      zerof)
            return carry2

        lax.fori_loop(0, C // (UNROLL * L), cbody, carry)
        return carry

    lax.fori_loop(0, RS, rbody, 0)


@functools.partial(
    pl.kernel,
    mesh=plsc.VectorSubcoreMesh(core_axis_name="c", subcore_axis_name="s"),
    out_type=jax.ShapeDtypeStruct((R, C), jnp.float32),
    scratch_types=[
        pltpu.VMEM((RS, C), jnp.float32),
        pltpu.VMEM((RS, C), jnp.float32),
        pltpu.VMEM((RS, C), jnp.float32),
        pltpu.VMEM((RS, C), jnp.float32),
        pltpu.SemaphoreType.DMA,
        pltpu.SemaphoreType.DMA,
        pltpu.SemaphoreType.DMA,
        pltpu.SemaphoreType.DMA,
    ],
    compiler_params=pltpu.CompilerParams(use_tc_tiling_on_sc=True),
)
def _nm_sparsity_sc(x_hbm, o_hbm, ib0, ib1, ob0, ob1, is0, is1, os0, os1):
    wid = lax.axis_index("s") * NC + lax.axis_index("c")
    base = wid * RPW
    ibufs, obufs = (ib0, ib1), (ob0, ob1)
    isems, osems = (is0, is1), (os0, os1)

    def start_in(s):
        b = s & 1
        pltpu.async_copy(
            x_hbm.at[pl.ds(base + s * RS, RS)], ibufs[b], isems[b])

    def start_out(s):
        b = s & 1
        pltpu.async_copy(
            obufs[b], o_hbm.at[pl.ds(base + s * RS, RS)], osems[b])

    def wait_in(b):
        pltpu.make_async_copy(
            x_hbm.at[pl.ds(base, RS)], ibufs[b], isems[b]).wait()

    def wait_out(b):
        pltpu.make_async_copy(
            obufs[b], o_hbm.at[pl.ds(base, RS)], osems[b]).wait()

    def pair(k, first, last):
        for par in (0, 1):
            s = 2 * k + par
            wait_in(par)
            if not first:
                wait_out(par)
            _compute_slab(ibufs[par], obufs[par])
            start_out_traced(s, par)
            if not last:
                start_in_traced(s + 2, par)

    def start_in_traced(s, par):
        pltpu.async_copy(
            x_hbm.at[pl.ds(base + s * RS, RS)], ibufs[par], isems[par])

    def start_out_traced(s, par):
        pltpu.async_copy(
            obufs[par], o_hbm.at[pl.ds(base + s * RS, RS)], osems[par])

    NPAIR = NSLAB // 2
    start_in(0)
    start_in(1)
    pair(0, first=True, last=False)

    def body(k, carry):
        pair(k, first=False, last=False)
        return carry

    lax.fori_loop(1, NPAIR - 1, body, 0)
    pair(NPAIR - 1, first=False, last=True)
    wait_out(0)
    wait_out(1)


def kernel(input):
    x = input.transpose(0, 2, 3, 1).reshape(R, C)
    out = _nm_sparsity_sc(x)
    return out.reshape(N, H, W, C).transpose(0, 3, 1, 2)
